# baseline (speedup 1.0000x reference)
import functools

import jax
import jax.numpy as jnp
from jax import lax
from jax.experimental import pallas as pl
from jax.experimental.pallas import tpu as pltpu
from jax.experimental.pallas import tpu_sc as plsc

N = 10000
D = 128

NC = 2
NS = 16
NW = NC * NS
CH = 128
NB = 8
NTAB = N
RPT = NTAB // NS
ZR = RPT // 5


NCHUNK = 2500
KPW = NCHUNK // NW
NEXTRA = NCHUNK - KPW * NW


def _sc_segment_sum():
    mesh = plsc.VectorSubcoreMesh(
        core_axis_name="c", subcore_axis_name="s", num_cores=NC, num_subcores=NS
    )

    @functools.partial(
        pl.kernel,
        out_type=jax.ShapeDtypeStruct((NC, NTAB, 16), jnp.float32),
        mesh=mesh,
        scratch_types=[
            pltpu.VMEM((KPW + 1, CH), jnp.int32),
            pltpu.VMEM((KPW + 1, CH), jnp.int32),
            pltpu.VMEM((2, NB, CH, 16), jnp.float32),
            pltpu.VMEM((ZR, 16), jnp.float32),
            pltpu.VMEM_SHARED((NTAB, 16), jnp.float32),
            pltpu.SemaphoreType.DMA,
            pltpu.SemaphoreType.DMA,
        ],
        compiler_params=pltpu.CompilerParams(use_tc_tiling_on_sc=False),
    )
    def k(tab_h, eidx_h, out_h, idx_s, idx_d, rows, zbuf, acc, sem_g, sem_s):
        c = lax.axis_index("c")
        s = lax.axis_index("s")
        w = s * NC + c
        z = jnp.zeros((16,), jnp.float32)
        for i in range(ZR):
            zbuf[i, :] = z
        base = s * RPT
        for r in range(RPT // ZR):
            pltpu.sync_copy(zbuf, acc.at[pl.ds(base + r * ZR, ZR)])
        pltpu.sync_copy(eidx_h.at[0, pl.ds(w * KPW, KPW)], idx_s.at[pl.ds(0, KPW)])
        pltpu.sync_copy(eidx_h.at[1, pl.ds(w * KPW, KPW)], idx_d.at[pl.ds(0, KPW)])
        extra = w < NEXTRA

        @pl.when(extra)
        def _():
            pltpu.sync_copy(eidx_h.at[0, NW * KPW + w], idx_s.at[KPW])
            pltpu.sync_copy(eidx_h.at[1, NW * KPW + w], idx_d.at[KPW])

        plsc.subcore_barrier()

        rounds = [range(r0, min(r0 + NB, KPW)) for r0 in range(0, KPW, NB)]

        def fire_gathers(chunks, grp):
            return [
                pltpu.async_copy(tab_h.at[idx_s.at[j]], rows.at[grp, b], sem_g)
                for b, j in enumerate(chunks)
            ]

        gd = fire_gathers(rounds[0], 0)
        for r, chunks in enumerate(rounds):
            grp = r % 2
            for d_ in gd:
                d_.wait()
            if r + 1 < len(rounds):
                gd = fire_gathers(rounds[r + 1], 1 - grp)
            sd = [
                pltpu.async_copy(rows.at[grp, b], acc.at[idx_d.at[j]], sem_s, add=True)
                for b, j in enumerate(chunks)
            ]
            for d_ in sd:
                d_.wait()

        @pl.when(extra)
        def _():
            pltpu.async_copy(tab_h.at[idx_s.at[KPW]], rows.at[0, 0], sem_g).wait()
            pltpu.async_copy(rows.at[0, 0], acc.at[idx_d.at[KPW]], sem_s, add=True).wait()

        plsc.subcore_barrier()
        pltpu.sync_copy(acc.at[pl.ds(base, RPT)], out_h.at[c, pl.ds(base, RPT)])

    return k


NR = NTAB // 8

import numpy as np


def _blk(b16):
    return jnp.kron(jnp.eye(8, dtype=jnp.float32), b16)


def _np_blk(b16):
    return np.kron(np.eye(8, dtype=np.float32), b16)


_P16 = np.zeros((16, 16), np.float32)
_P16[8, 0:8] = 1.0
_PCNT = _np_blk(_P16)
_R16 = np.zeros((16, 16), np.float32)
_R16[8:16, 0:8] = np.eye(8, dtype=np.float32)
_PRES = _np_blk(_R16)
_S16 = np.zeros((16, 16), np.float32)
_S16[0, 1] = 1.0
_S16[1, 0] = 1.0
_PSWAP = _np_blk(_S16)
_POUT = np.zeros((128, 16), np.float32)
for _p in range(8):
    _POUT[16 * _p, 2 * _p] = 1.0
    _POUT[16 * _p + 1, 2 * _p + 1] = 1.0


def _onehot8(ncols):
    lane = lax.broadcasted_iota(jnp.int32, (NR, ncols), 1)
    return (lane % 16 == 8).astype(jnp.float32)


def _tc_pre(xflat, wbt, wbs, bbt):
    def body(x_ref, wt_ref, ws_ref, b_ref, t_ref, s_ref):
        x = x_ref[...]
        u = jnp.dot(x, wt_ref[...], preferred_element_type=jnp.float32)
        t_ref[...] = u + _onehot8(128)
        s_ref[...] = jnp.dot(x, ws_ref[...], preferred_element_type=jnp.float32) + b_ref[...]

    return pl.pallas_call(
        body,
        out_shape=[
            jax.ShapeDtypeStruct((NR, 128), jnp.float32),
            jax.ShapeDtypeStruct((NR, 128), jnp.float32),
        ],
    )(xflat, wbt, wbs, bbt)


def _tc_mid(a, sprev, wtblk, wsblk, bbt, pcnt, pres):
    def body(a_ref, s_ref, wt_ref, ws_ref, b_ref, pc_ref, pr_ref, t_ref, so_ref):
        agg = a_ref[0] + a_ref[1]
        recip = 1.0 / jnp.maximum(agg, 1.0)
        rb = jnp.dot(recip, pc_ref[...], preferred_element_type=jnp.float32)
        s = s_ref[...]
        h = jnp.maximum(agg * rb + s, 0.0)
        x1 = h + jnp.dot(s, pr_ref[...], preferred_element_type=jnp.float32)
        u = jnp.dot(x1, wt_ref[...], preferred_element_type=jnp.float32)
        t_ref[...] = u + _onehot8(128)
        so_ref[...] = jnp.dot(x1, ws_ref[...], preferred_element_type=jnp.float32) + b_ref[...]

    return pl.pallas_call(
        body,
        out_shape=[
            jax.ShapeDtypeStruct((NR, 128), jnp.float32),
            jax.ShapeDtypeStruct((NR, 128), jnp.float32),
        ],
    )(a, sprev, wtblk, wsblk, bbt, pcnt, pres)


def _tc_final(a, sprev, pcnt, pswap, pout):
    def body(a_ref, s_ref, pc_ref, psw_ref, po_ref, o_ref):
        agg = a_ref[0] + a_ref[1]
        recip = 1.0 / jnp.maximum(agg, 1.0)
        rb = jnp.dot(recip, pc_ref[...], preferred_element_type=jnp.float32)
        x3 = agg * rb + s_ref[...]
        sw = jnp.dot(x3, psw_ref[...], preferred_element_type=jnp.float32)
        mx = jnp.maximum(x3, sw)
        lse = mx + jnp.log(jnp.exp(x3 - mx) + jnp.exp(sw - mx))
        o_ref[...] = jnp.dot(
            x3 - lse, po_ref[...], preferred_element_type=jnp.float32
        )

    return pl.pallas_call(
        body,
        out_shape=jax.ShapeDtypeStruct((NR, 16), jnp.float32),
    )(a, sprev, pcnt, pswap, pout)


def kernel(x, edge_index, Wl1, Wr1, b1, Wl2, Wr2, b2, Wl3, Wr3, b3, R1, rb1, R2, rb2):
    eidx = edge_index.reshape(2, NCHUNK, CH)

    xflat = x.reshape(NR, 8 * D)
    eye8 = jnp.eye(8, dtype=jnp.float32)
    z8 = jnp.zeros((D, 8), jnp.float32)
    wbt1 = jnp.kron(eye8, jnp.concatenate([Wl1, z8], axis=1))
    wbs1 = jnp.kron(eye8, jnp.concatenate([Wr1, R1], axis=1))
    bbt1 = jnp.tile(jnp.concatenate([b1, rb1]), 8).reshape(1, 128)

    z16 = jnp.zeros((16, 16), jnp.float32)
    wt2 = _blk(z16.at[:8, :8].set(Wl2))
    ws2 = _blk(z16.at[:8, :8].set(Wr2).at[:8, 8:16].set(R2))
    bbt2 = jnp.tile(jnp.concatenate([b2, rb2]), 8).reshape(1, 128)
    wt3 = _blk(z16.at[:8, :2].set(Wl3))
    ws3 = _blk(z16.at[:8, :2].set(Wr3))
    bbt3 = jnp.tile(
        jnp.concatenate([b3, jnp.zeros((14,), jnp.float32)]), 8
    ).reshape(1, 128)

    pcnt = jnp.asarray(_PCNT)
    pres = jnp.asarray(_PRES)
    pswap = jnp.asarray(_PSWAP)
    pout = jnp.asarray(_POUT)

    sc = _sc_segment_sum()
    t1, s1 = _tc_pre(xflat, wbt1, wbs1, bbt1)
    a1 = sc(t1.reshape(NTAB, 16), eidx)
    t2, s2 = _tc_mid(a1.reshape(NC, NR, 128), s1, wt2, ws2, bbt2, pcnt, pres)
    a2 = sc(t2.reshape(NTAB, 16), eidx)
    t3, s3 = _tc_mid(a2.reshape(NC, NR, 128), s2, wt3, ws3, bbt3, pcnt, pres)
    a3 = sc(t3.reshape(NTAB, 16), eidx)
    out = _tc_final(a3.reshape(NC, NR, 128), s3, pcnt, pswap, pout)
    return out.reshape(N, 2)


# speedup vs baseline: 1.0559x; 1.0559x over previous
import functools

import jax
import jax.numpy as jnp
from jax import lax
from jax.experimental import pallas as pl
from jax.experimental.pallas import tpu as pltpu
from jax.experimental.pallas import tpu_sc as plsc

N = 10000
D = 128

NC = 2
NS = 16
NW = NC * NS
CH = 128
NB = 8
NTAB = 10048
RPT = NTAB // NS
ZR = RPT // 4


NCHUNK = 2500
KPW = NCHUNK // NW
NEXTRA = NCHUNK - KPW * NW


def _sc_segment_sum():
    mesh = plsc.VectorSubcoreMesh(
        core_axis_name="c", subcore_axis_name="s", num_cores=NC, num_subcores=NS
    )

    @functools.partial(
        pl.kernel,
        out_type=jax.ShapeDtypeStruct((NC, NTAB, 16), jnp.float32),
        mesh=mesh,
        scratch_types=[
            pltpu.VMEM((KPW + 1, CH), jnp.int32),
            pltpu.VMEM((KPW + 1, CH), jnp.int32),
            pltpu.VMEM((2, NB, CH, 16), jnp.float32),
            pltpu.VMEM((ZR, 16), jnp.float32),
            pltpu.VMEM_SHARED((NTAB, 16), jnp.float32),
            pltpu.SemaphoreType.DMA,
            pltpu.SemaphoreType.DMA,
        ],
        compiler_params=pltpu.CompilerParams(use_tc_tiling_on_sc=False),
    )
    def k(tab_h, eidx_h, out_h, idx_s, idx_d, rows, zbuf, acc, sem_g, sem_s):
        c = lax.axis_index("c")
        s = lax.axis_index("s")
        w = s * NC + c
        z = jnp.zeros((16,), jnp.float32)
        for i in range(ZR):
            zbuf[i, :] = z
        base = s * RPT
        for r in range(RPT // ZR):
            pltpu.sync_copy(zbuf, acc.at[pl.ds(base + r * ZR, ZR)])
        pltpu.sync_copy(eidx_h.at[0, pl.ds(w * KPW, KPW)], idx_s.at[pl.ds(0, KPW)])
        pltpu.sync_copy(eidx_h.at[1, pl.ds(w * KPW, KPW)], idx_d.at[pl.ds(0, KPW)])
        extra = w < NEXTRA

        @pl.when(extra)
        def _():
            pltpu.sync_copy(eidx_h.at[0, NW * KPW + w], idx_s.at[KPW])
            pltpu.sync_copy(eidx_h.at[1, NW * KPW + w], idx_d.at[KPW])

        plsc.subcore_barrier()

        rounds = [range(r0, min(r0 + NB, KPW)) for r0 in range(0, KPW, NB)]

        def fire_gathers(chunks, grp):
            return [
                pltpu.async_copy(tab_h.at[idx_s.at[j]], rows.at[grp, b], sem_g)
                for b, j in enumerate(chunks)
            ]

        gd = fire_gathers(rounds[0], 0)
        for r, chunks in enumerate(rounds):
            grp = r % 2
            for d_ in gd:
                d_.wait()
            if r + 1 < len(rounds):
                gd = fire_gathers(rounds[r + 1], 1 - grp)
            sd = [
                pltpu.async_copy(rows.at[grp, b], acc.at[idx_d.at[j]], sem_s, add=True)
                for b, j in enumerate(chunks)
            ]
            for d_ in sd:
                d_.wait()

        @pl.when(extra)
        def _():
            pltpu.async_copy(tab_h.at[idx_s.at[KPW]], rows.at[0, 0], sem_g).wait()
            pltpu.async_copy(rows.at[0, 0], acc.at[idx_d.at[KPW]], sem_s, add=True).wait()

        plsc.subcore_barrier()
        pltpu.sync_copy(acc.at[pl.ds(base, RPT)], out_h.at[c, pl.ds(base, RPT)])

    return k


NR = NTAB // 8
NLAST = N - 7 * NR

import numpy as np


def _blk(b16):
    return jnp.kron(jnp.eye(8, dtype=jnp.float32), b16)


def _np_blk(b16):
    return np.kron(np.eye(8, dtype=np.float32), b16)


_P16 = np.zeros((16, 16), np.float32)
_P16[8, 0:8] = 1.0
_PCNT = _np_blk(_P16)
_R16 = np.zeros((16, 16), np.float32)
_R16[8:16, 0:8] = np.eye(8, dtype=np.float32)
_PRES = _np_blk(_R16)
_S16 = np.zeros((16, 16), np.float32)
_S16[0, 1] = 1.0
_S16[1, 0] = 1.0
_PSWAP = _np_blk(_S16)
_POUT = np.zeros((128, 16), np.float32)
for _p in range(8):
    _POUT[16 * _p, 2 * _p] = 1.0
    _POUT[16 * _p + 1, 2 * _p + 1] = 1.0


def _onehot8(nrows, ncols):
    lane = lax.broadcasted_iota(jnp.int32, (nrows, ncols), 1)
    return (lane % 16 == 8).astype(jnp.float32)


def _tc_pre(x, w, bb):
    def body(x_ref, w_ref, b_ref, t_ref, s_ref):
        for p in range(8):
            nrows = NR if p < 7 else NLAST
            u = jnp.dot(
                x_ref[pl.ds(NR * p, nrows), :],
                w_ref[...],
                preferred_element_type=jnp.float32,
            )
            sl = pl.ds(16 * p, 16)
            t_ref[pl.ds(0, nrows), sl] = u[:, :16] + _onehot8(nrows, 16)
            s_ref[pl.ds(0, nrows), sl] = u[:, 16:32] + b_ref[...]
        z = jnp.zeros((NR - NLAST, 16), jnp.float32)
        t_ref[pl.ds(NLAST, NR - NLAST), pl.ds(112, 16)] = z
        s_ref[pl.ds(NLAST, NR - NLAST), pl.ds(112, 16)] = z

    return pl.pallas_call(
        body,
        out_shape=[
            jax.ShapeDtypeStruct((NR, 128), jnp.float32),
            jax.ShapeDtypeStruct((NR, 128), jnp.float32),
        ],
    )(x, w, bb)


def _tc_mid(a, sprev, wtblk, wsblk, bbt, pcnt, pres):
    def body(a_ref, s_ref, wt_ref, ws_ref, b_ref, pc_ref, pr_ref, t_ref, so_ref):
        agg = a_ref[0] + a_ref[1]
        recip = 1.0 / jnp.maximum(agg, 1.0)
        rb = jnp.dot(recip, pc_ref[...], preferred_element_type=jnp.float32)
        s = s_ref[...]
        h = jnp.maximum(agg * rb + s, 0.0)
        x1 = h + jnp.dot(s, pr_ref[...], preferred_element_type=jnp.float32)
        u = jnp.dot(x1, wt_ref[...], preferred_element_type=jnp.float32)
        t_ref[...] = u + _onehot8(NR, 128)
        so_ref[...] = jnp.dot(x1, ws_ref[...], preferred_element_type=jnp.float32) + b_ref[...]

    return pl.pallas_call(
        body,
        out_shape=[
            jax.ShapeDtypeStruct((NR, 128), jnp.float32),
            jax.ShapeDtypeStruct((NR, 128), jnp.float32),
        ],
    )(a, sprev, wtblk, wsblk, bbt, pcnt, pres)


def _tc_final(a, sprev, pcnt, pswap, pout):
    def body(a_ref, s_ref, pc_ref, psw_ref, po_ref, o_ref):
        agg = a_ref[0] + a_ref[1]
        recip = 1.0 / jnp.maximum(agg, 1.0)
        rb = jnp.dot(recip, pc_ref[...], preferred_element_type=jnp.float32)
        x3 = agg * rb + s_ref[...]
        sw = jnp.dot(x3, psw_ref[...], preferred_element_type=jnp.float32)
        mx = jnp.maximum(x3, sw)
        lse = mx + jnp.log(jnp.exp(x3 - mx) + jnp.exp(sw - mx))
        o_ref[...] = jnp.dot(
            x3 - lse, po_ref[...], preferred_element_type=jnp.float32
        )

    return pl.pallas_call(
        body,
        out_shape=jax.ShapeDtypeStruct((NR, 16), jnp.float32),
    )(a, sprev, pcnt, pswap, pout)


def kernel(x, edge_index, Wl1, Wr1, b1, Wl2, Wr2, b2, Wl3, Wr3, b3, R1, rb1, R2, rb2):
    tau = (edge_index % NR) * 8 + edge_index // NR
    eidx = tau.reshape(2, NCHUNK, CH)

    z8 = jnp.zeros((D, 8), jnp.float32)
    w1 = jnp.concatenate([Wl1, z8, Wr1, R1], axis=1)
    bb1 = jnp.concatenate([b1, rb1]).reshape(1, 16)

    z16 = jnp.zeros((16, 16), jnp.float32)
    wt2 = _blk(z16.at[:8, :8].set(Wl2))
    ws2 = _blk(z16.at[:8, :8].set(Wr2).at[:8, 8:16].set(R2))
    bbt2 = jnp.tile(jnp.concatenate([b2, rb2]), 8).reshape(1, 128)
    wt3 = _blk(z16.at[:8, :2].set(Wl3))
    ws3 = _blk(z16.at[:8, :2].set(Wr3))
    bbt3 = jnp.tile(
        jnp.concatenate([b3, jnp.zeros((14,), jnp.float32)]), 8
    ).reshape(1, 128)

    pcnt = jnp.asarray(_PCNT)
    pres = jnp.asarray(_PRES)
    pswap = jnp.asarray(_PSWAP)
    pout = jnp.asarray(_POUT)

    sc = _sc_segment_sum()
    t1, s1 = _tc_pre(x, w1, bb1)
    a1 = sc(t1.reshape(NTAB, 16), eidx)
    t2, s2 = _tc_mid(a1.reshape(NC, NR, 128), s1, wt2, ws2, bbt2, pcnt, pres)
    a2 = sc(t2.reshape(NTAB, 16), eidx)
    t3, s3 = _tc_mid(a2.reshape(NC, NR, 128), s2, wt3, ws3, bbt3, pcnt, pres)
    a3 = sc(t3.reshape(NTAB, 16), eidx)
    out = _tc_final(a3.reshape(NC, NR, 128), s3, pcnt, pswap, pout)
    return out.reshape(NR, 8, 2).transpose(1, 0, 2).reshape(NTAB, 2)[:N]


# speedup vs baseline: 1.1727x; 1.1106x over previous
import functools

import jax
import jax.numpy as jnp
from jax import lax
from jax.experimental import pallas as pl
from jax.experimental.pallas import tpu as pltpu
from jax.experimental.pallas import tpu_sc as plsc

N = 10000
D = 128

NC = 2
NS = 16
NW = NC * NS
CH = 128
NB = 8
NTAB = 10048
RPT = NTAB // NS
ZR = RPT // 4


NCHUNK = 2500
KPW = NCHUNK // NW
NEXTRA = NCHUNK - KPW * NW


def _sc_segment_sum():
    mesh = plsc.VectorSubcoreMesh(
        core_axis_name="c", subcore_axis_name="s", num_cores=NC, num_subcores=NS
    )

    @functools.partial(
        pl.kernel,
        out_type=jax.ShapeDtypeStruct((NC, NTAB, 16), jnp.float32),
        mesh=mesh,
        scratch_types=[
            pltpu.VMEM((KPW + 1, CH), jnp.int32),
            pltpu.VMEM((KPW + 1, CH), jnp.int32),
            pltpu.VMEM((2, NB, CH, 16), jnp.float32),
            pltpu.VMEM((ZR, 16), jnp.float32),
            pltpu.VMEM_SHARED((NTAB, 16), jnp.float32),
            pltpu.VMEM_SHARED((NTAB, 16), jnp.float32),
            pltpu.SemaphoreType.DMA,
            pltpu.SemaphoreType.DMA,
        ],
        compiler_params=pltpu.CompilerParams(use_tc_tiling_on_sc=False),
    )
    def k(tab_h, eidx_h, out_h, idx_s, idx_d, rows, zbuf, acc, tab_s, sem_g, sem_s):
        c = lax.axis_index("c")
        s = lax.axis_index("s")
        w = s * NC + c
        z = jnp.zeros((16,), jnp.float32)
        for i in range(ZR):
            zbuf[i, :] = z
        base = s * RPT
        for r in range(RPT // ZR):
            pltpu.sync_copy(zbuf, acc.at[pl.ds(base + r * ZR, ZR)])
        pltpu.sync_copy(tab_h.at[pl.ds(base, RPT)], tab_s.at[pl.ds(base, RPT)])
        pltpu.sync_copy(eidx_h.at[0, pl.ds(w * KPW, KPW)], idx_s.at[pl.ds(0, KPW)])
        pltpu.sync_copy(eidx_h.at[1, pl.ds(w * KPW, KPW)], idx_d.at[pl.ds(0, KPW)])
        extra = w < NEXTRA

        @pl.when(extra)
        def _():
            pltpu.sync_copy(eidx_h.at[0, NW * KPW + w], idx_s.at[KPW])
            pltpu.sync_copy(eidx_h.at[1, NW * KPW + w], idx_d.at[KPW])

        plsc.subcore_barrier()

        rounds = [range(r0, min(r0 + NB, KPW)) for r0 in range(0, KPW, NB)]

        def fire_gathers(chunks, grp):
            return [
                pltpu.async_copy(tab_s.at[idx_s.at[j]], rows.at[grp, b], sem_g)
                for b, j in enumerate(chunks)
            ]

        gd = fire_gathers(rounds[0], 0)
        for r, chunks in enumerate(rounds):
            grp = r % 2
            for d_ in gd:
                d_.wait()
            if r + 1 < len(rounds):
                gd = fire_gathers(rounds[r + 1], 1 - grp)
            sd = [
                pltpu.async_copy(rows.at[grp, b], acc.at[idx_d.at[j]], sem_s, add=True)
                for b, j in enumerate(chunks)
            ]
            for d_ in sd:
                d_.wait()

        @pl.when(extra)
        def _():
            pltpu.async_copy(tab_s.at[idx_s.at[KPW]], rows.at[0, 0], sem_g).wait()
            pltpu.async_copy(rows.at[0, 0], acc.at[idx_d.at[KPW]], sem_s, add=True).wait()

        plsc.subcore_barrier()
        pltpu.sync_copy(acc.at[pl.ds(base, RPT)], out_h.at[c, pl.ds(base, RPT)])

    return k


NR = NTAB // 8
NLAST = N - 7 * NR

import numpy as np


def _blk(b16):
    return jnp.kron(jnp.eye(8, dtype=jnp.float32), b16)


def _np_blk(b16):
    return np.kron(np.eye(8, dtype=np.float32), b16)


_P16 = np.zeros((16, 16), np.float32)
_P16[8, 0:8] = 1.0
_PCNT = _np_blk(_P16)
_R16 = np.zeros((16, 16), np.float32)
_R16[8:16, 0:8] = np.eye(8, dtype=np.float32)
_PRES = _np_blk(_R16)
_S16 = np.zeros((16, 16), np.float32)
_S16[0, 1] = 1.0
_S16[1, 0] = 1.0
_PSWAP = _np_blk(_S16)
_POUT = np.zeros((128, 16), np.float32)
for _p in range(8):
    _POUT[16 * _p, 2 * _p] = 1.0
    _POUT[16 * _p + 1, 2 * _p + 1] = 1.0


def _onehot8(nrows, ncols):
    lane = lax.broadcasted_iota(jnp.int32, (nrows, ncols), 1)
    return (lane % 16 == 8).astype(jnp.float32)


def _tc_pre(x, w, bb):
    def body(x_ref, w_ref, b_ref, t_ref, s_ref):
        for p in range(8):
            nrows = NR if p < 7 else NLAST
            u = jnp.dot(
                x_ref[pl.ds(NR * p, nrows), :],
                w_ref[...],
                preferred_element_type=jnp.float32,
            )
            sl = pl.ds(16 * p, 16)
            t_ref[pl.ds(0, nrows), sl] = u[:, :16] + _onehot8(nrows, 16)
            s_ref[pl.ds(0, nrows), sl] = u[:, 16:32] + b_ref[...]
        z = jnp.zeros((NR - NLAST, 16), jnp.float32)
        t_ref[pl.ds(NLAST, NR - NLAST), pl.ds(112, 16)] = z
        s_ref[pl.ds(NLAST, NR - NLAST), pl.ds(112, 16)] = z

    return pl.pallas_call(
        body,
        out_shape=[
            jax.ShapeDtypeStruct((NR, 128), jnp.float32),
            jax.ShapeDtypeStruct((NR, 128), jnp.float32),
        ],
    )(x, w, bb)


def _tc_mid(a, sprev, wtblk, wsblk, bbt, pcnt, pres):
    def body(a_ref, s_ref, wt_ref, ws_ref, b_ref, pc_ref, pr_ref, t_ref, so_ref):
        agg = a_ref[0] + a_ref[1]
        recip = 1.0 / jnp.maximum(agg, 1.0)
        rb = jnp.dot(recip, pc_ref[...], preferred_element_type=jnp.float32)
        s = s_ref[...]
        h = jnp.maximum(agg * rb + s, 0.0)
        x1 = h + jnp.dot(s, pr_ref[...], preferred_element_type=jnp.float32)
        u = jnp.dot(x1, wt_ref[...], preferred_element_type=jnp.float32)
        t_ref[...] = u + _onehot8(NR, 128)
        so_ref[...] = jnp.dot(x1, ws_ref[...], preferred_element_type=jnp.float32) + b_ref[...]

    return pl.pallas_call(
        body,
        out_shape=[
            jax.ShapeDtypeStruct((NR, 128), jnp.float32),
            jax.ShapeDtypeStruct((NR, 128), jnp.float32),
        ],
    )(a, sprev, wtblk, wsblk, bbt, pcnt, pres)


def _tc_final(a, sprev, pcnt, pswap, pout):
    def body(a_ref, s_ref, pc_ref, psw_ref, po_ref, o_ref):
        agg = a_ref[0] + a_ref[1]
        recip = 1.0 / jnp.maximum(agg, 1.0)
        rb = jnp.dot(recip, pc_ref[...], preferred_element_type=jnp.float32)
        x3 = agg * rb + s_ref[...]
        sw = jnp.dot(x3, psw_ref[...], preferred_element_type=jnp.float32)
        mx = jnp.maximum(x3, sw)
        lse = mx + jnp.log(jnp.exp(x3 - mx) + jnp.exp(sw - mx))
        o_ref[...] = jnp.dot(
            x3 - lse, po_ref[...], preferred_element_type=jnp.float32
        )

    return pl.pallas_call(
        body,
        out_shape=jax.ShapeDtypeStruct((NR, 16), jnp.float32),
    )(a, sprev, pcnt, pswap, pout)


def kernel(x, edge_index, Wl1, Wr1, b1, Wl2, Wr2, b2, Wl3, Wr3, b3, R1, rb1, R2, rb2):
    tau = (edge_index % NR) * 8 + edge_index // NR
    eidx = tau.reshape(2, NCHUNK, CH)

    z8 = jnp.zeros((D, 8), jnp.float32)
    w1 = jnp.concatenate([Wl1, z8, Wr1, R1], axis=1)
    bb1 = jnp.concatenate([b1, rb1]).reshape(1, 16)

    z16 = jnp.zeros((16, 16), jnp.float32)
    wt2 = _blk(z16.at[:8, :8].set(Wl2))
    ws2 = _blk(z16.at[:8, :8].set(Wr2).at[:8, 8:16].set(R2))
    bbt2 = jnp.tile(jnp.concatenate([b2, rb2]), 8).reshape(1, 128)
    wt3 = _blk(z16.at[:8, :2].set(Wl3))
    ws3 = _blk(z16.at[:8, :2].set(Wr3))
    bbt3 = jnp.tile(
        jnp.concatenate([b3, jnp.zeros((14,), jnp.float32)]), 8
    ).reshape(1, 128)

    pcnt = jnp.asarray(_PCNT)
    pres = jnp.asarray(_PRES)
    pswap = jnp.asarray(_PSWAP)
    pout = jnp.asarray(_POUT)

    sc = _sc_segment_sum()
    t1, s1 = _tc_pre(x, w1, bb1)
    a1 = sc(t1.reshape(NTAB, 16), eidx)
    t2, s2 = _tc_mid(a1.reshape(NC, NR, 128), s1, wt2, ws2, bbt2, pcnt, pres)
    a2 = sc(t2.reshape(NTAB, 16), eidx)
    t3, s3 = _tc_mid(a2.reshape(NC, NR, 128), s2, wt3, ws3, bbt3, pcnt, pres)
    a3 = sc(t3.reshape(NTAB, 16), eidx)
    out = _tc_final(a3.reshape(NC, NR, 128), s3, pcnt, pswap, pout)
    return out.reshape(NR, 8, 2).transpose(1, 0, 2).reshape(NTAB, 2)[:N]
